# baseline (speedup 1.0000x reference)
import functools

import jax
import jax.numpy as jnp
from jax import lax
from jax.experimental import pallas as pl
from jax.experimental.pallas import tpu as pltpu
from jax.experimental.pallas import tpu_sc as plsc

_SC_CORES = 2
_SC_SUBCORES = 16
_NW = _SC_CORES * _SC_SUBCORES

_TM = 256


def _router_meta_body(x_ref, rw_ref, topw_ref, p0_ref, p1_ref, ta_ref,
                      idx_scr, rank_scr, carry_scr):
    ph = pl.program_id(0)
    i = pl.program_id(1)
    nb = pl.num_programs(1)
    tb = x_ref.shape[0]
    e = rw_ref.shape[0]

    @pl.when((ph == 0) & (i == 0))
    def _():
        carry_scr[...] = jnp.zeros_like(carry_scr)

    @pl.when(ph == 0)
    def _():
        xb = x_ref[...]
        logits = lax.dot_general(
            xb, rw_ref[...], (((1,), (1,)), ((), ())),
            preferred_element_type=jnp.float32)
        m = jnp.max(logits, axis=-1, keepdims=True)
        p = jnp.exp(logits - m)
        p = p / jnp.sum(p, axis=-1, keepdims=True)
        ids = lax.broadcasted_iota(jnp.int32, p.shape, 1)
        m1 = jnp.max(p, axis=-1, keepdims=True)
        i1 = jnp.min(jnp.where(p == m1, ids, e), axis=-1, keepdims=True)
        pm = jnp.where(ids == i1, -1.0, p)
        m2 = jnp.max(pm, axis=-1, keepdims=True)
        i2 = jnp.min(jnp.where(pm == m2, ids, e), axis=-1, keepdims=True)
        s = m1 + m2
        topw_ref[...] = jnp.concatenate([m1 / s, m2 / s], axis=1)
        oh = ((ids == i1) | (ids == i2)).astype(jnp.float32)
        r = lax.broadcasted_iota(jnp.int32, (tb, tb), 0)
        c = lax.broadcasted_iota(jnp.int32, (tb, tb), 1)
        tril = (r > c).astype(jnp.float32)
        pref = lax.dot_general(
            tril, oh, (((1,), (0,)), ((), ())),
            preferred_element_type=jnp.float32)
        prefc = pref + carry_scr[...]
        rank0 = jnp.sum(jnp.where(ids == i1, prefc, 0.0), axis=1,
                        keepdims=True)
        rank1 = jnp.sum(jnp.where(ids == i2, prefc, 0.0), axis=1,
                        keepdims=True)
        idx_scr[pl.ds(i * tb, tb), :] = jnp.concatenate([i1, i2], axis=1)
        rank_scr[pl.ds(i * tb, tb), :] = jnp.concatenate([rank0, rank1],
                                                         axis=1)
        carry_scr[...] += jnp.sum(oh, axis=0, keepdims=True)

    @pl.when(ph == 1)
    def _():
        counts = carry_scr[...]
        padded = jnp.ceil(counts / _TM) * _TM
        le = (lax.broadcasted_iota(jnp.int32, (e, e), 0)
              <= lax.broadcasted_iota(jnp.int32, (e, e), 1)
              ).astype(jnp.float32)
        pend = lax.dot_general(
            padded, le, (((1,), (0,)), ((), ())),
            preferred_element_type=jnp.float32)
        pstart = pend - padded
        tpi = idx_scr[pl.ds(i * tb, tb), :]
        rk = rank_scr[pl.ds(i * tb, tb), :]
        ids = lax.broadcasted_iota(jnp.int32, (tb, e), 1)
        ps0 = jnp.sum(jnp.where(ids == tpi[:, 0:1], pstart, 0.0), axis=1,
                      keepdims=True)
        ps1 = jnp.sum(jnp.where(ids == tpi[:, 1:2], pstart, 0.0), axis=1,
                      keepdims=True)
        p0_ref[...] = (ps0 + rk[:, 0:1]).astype(jnp.int32)
        p1_ref[...] = (ps1 + rk[:, 1:2]).astype(jnp.int32)

        @pl.when(i == nb - 1)
        def _():
            ntt = ta_ref.shape[0]
            basev = (lax.broadcasted_iota(jnp.int32, (ntt, 1), 0)
                     * _TM).astype(jnp.float32)
            cmp = pend <= basev
            te = jnp.minimum(jnp.sum(cmp.astype(jnp.int32), axis=1,
                                     keepdims=True), e - 1)
            act = (basev < jnp.max(pend)).astype(jnp.int32)
            ta_ref[...] = jnp.concatenate([te, act], axis=1)


def _router_meta(x_flat, router_w, nt):
    n, d = x_flat.shape
    e = router_w.shape[0]
    tb = 512
    nb = n // tb
    return pl.pallas_call(
        _router_meta_body,
        grid=(2, nb),
        in_specs=[
            pl.BlockSpec((tb, d), lambda p, i: ((1 - p) * i, 0)),
            pl.BlockSpec((e, d), lambda p, i: (0, 0)),
        ],
        out_specs=[
            pl.BlockSpec((tb, 2), lambda p, i, _nb=nb: ((1 - p) * i + p * (_nb - 1), 0)),
            pl.BlockSpec((tb, 1), lambda p, i: (p * i, 0)),
            pl.BlockSpec((tb, 1), lambda p, i: (p * i, 0)),
            pl.BlockSpec((nt, 2), lambda p, i: (0, 0)),
        ],
        out_shape=[
            jax.ShapeDtypeStruct((n, 2), jnp.float32),
            jax.ShapeDtypeStruct((n, 1), jnp.int32),
            jax.ShapeDtypeStruct((n, 1), jnp.int32),
            jax.ShapeDtypeStruct((nt, 2), jnp.int32),
        ],
        scratch_shapes=[
            pltpu.VMEM((n, 2), jnp.int32),
            pltpu.VMEM((n, 2), jnp.float32),
            pltpu.VMEM((1, e), jnp.float32),
        ],
        compiler_params=pltpu.CompilerParams(
            dimension_semantics=("arbitrary", "arbitrary"),
        ),
    )(x_flat, router_w)


def _make_sc_gather(rows_out, d, table_rows):
    dw = d // 2
    per_w = rows_out // _NW
    ch = per_w
    while ch > 96 or per_w % ch:
        ch -= 8
    n_ch = per_w // ch
    mesh = plsc.VectorSubcoreMesh(
        core_axis_name="c", subcore_axis_name="s",
        num_cores=_SC_CORES, num_subcores=_SC_SUBCORES,
    )

    @functools.partial(
        pl.kernel,
        out_type=jax.ShapeDtypeStruct((rows_out, dw), jnp.int32),
        mesh=mesh,
        scratch_types=[
            pltpu.VMEM((2, ch), jnp.int32),
            pltpu.VMEM((2, ch, dw), jnp.int32),
            pltpu.SemaphoreType.DMA((2,)),
        ],
    )
    def gather(table_hbm, idx_hbm, out_hbm, idx_v, rows_v, sems):
        wid = lax.axis_index("s") * _SC_CORES + lax.axis_index("c")
        base = wid * per_w

        def start(c):
            b = c % 2
            pltpu.sync_copy(idx_hbm.at[pl.ds(base + c * ch, ch)], idx_v.at[b])
            return pltpu.async_copy(table_hbm.at[idx_v.at[b]], rows_v.at[b],
                                    sems.at[b])

        handles = [start(0)]
        for c in range(n_ch):
            if c + 1 < n_ch:
                handles.append(start(c + 1))
            handles[c].wait()
            pltpu.sync_copy(rows_v.at[c % 2],
                            out_hbm.at[pl.ds(base + c * ch, ch)])

    return gather


def _make_sc_dispatch(n, d, np_):
    dw = d // 2
    per_w = n // _NW
    ch = per_w
    while ch > 96 or per_w % ch:
        ch -= 8
    n_ch = per_w // ch
    mesh = plsc.VectorSubcoreMesh(
        core_axis_name="c", subcore_axis_name="s",
        num_cores=_SC_CORES, num_subcores=_SC_SUBCORES,
    )

    @functools.partial(
        pl.kernel,
        out_type=jax.ShapeDtypeStruct((np_, dw), jnp.int32),
        mesh=mesh,
        scratch_types=[
            pltpu.VMEM((2, ch), jnp.int32),
            pltpu.VMEM((2, ch), jnp.int32),
            pltpu.VMEM((2, ch, dw), jnp.int32),
            pltpu.SemaphoreType.DMA((2,)),
        ],
    )
    def dispatch(x_hbm, p0_hbm, p1_hbm, out_hbm, i0_v, i1_v, rows_v, sems):
        wid = lax.axis_index("s") * _SC_CORES + lax.axis_index("c")
        base = wid * per_w

        def start(c):
            b = c % 2
            sl = pl.ds(base + c * ch, ch)
            pltpu.sync_copy(x_hbm.at[sl], rows_v.at[b])
            pltpu.sync_copy(p0_hbm.at[sl], i0_v.at[b])
            pltpu.sync_copy(p1_hbm.at[sl], i1_v.at[b])
            h0 = pltpu.async_copy(rows_v.at[b], out_hbm.at[i0_v.at[b]],
                                  sems.at[b])
            h1 = pltpu.async_copy(rows_v.at[b], out_hbm.at[i1_v.at[b]],
                                  sems.at[b])
            return h0, h1

        handles = [start(0)]
        for c in range(n_ch):
            if c + 1 < n_ch:
                handles.append(start(c + 1))
            h0, h1 = handles[c]
            h0.wait()
            h1.wait()

    return dispatch


def _ffn_body(te_ref, act_ref, xs_ref, w1_ref, w3_ref, w2_ref, out_ref):
    i = pl.program_id(0)

    @pl.when(act_ref[i] != 0)
    def _():
        xb = xs_ref[...]
        h1 = lax.dot_general(
            xb, w1_ref[0], (((1,), (1,)), ((), ())),
            preferred_element_type=jnp.float32)
        h3 = lax.dot_general(
            xb, w3_ref[0], (((1,), (1,)), ((), ())),
            preferred_element_type=jnp.float32)
        g = ((h1 * jax.nn.sigmoid(h1)) * h3).astype(jnp.bfloat16)
        out_ref[...] = lax.dot_general(
            g, w2_ref[0], (((1,), (1,)), ((), ())),
            preferred_element_type=jnp.float32).astype(jnp.bfloat16)


def _ffn(xs, w13, w2, te, act):
    np_, d = xs.shape
    e, f2, _ = w13.shape
    f = f2 // 2
    nt = np_ // _TM
    grid_spec = pltpu.PrefetchScalarGridSpec(
        num_scalar_prefetch=2,
        grid=(nt,),
        in_specs=[
            pl.BlockSpec((_TM, d), lambda i, te, act: (i, 0)),
            pl.BlockSpec((1, f, d), lambda i, te, act: (te[i], 0, 0)),
            pl.BlockSpec((1, f, d), lambda i, te, act: (te[i], 1, 0)),
            pl.BlockSpec((1, d, f), lambda i, te, act: (te[i], 0, 0)),
        ],
        out_specs=pl.BlockSpec((_TM, d), lambda i, te, act: (i, 0)),
    )
    return pl.pallas_call(
        _ffn_body,
        grid_spec=grid_spec,
        out_shape=jax.ShapeDtypeStruct((np_, d), jnp.bfloat16),
        compiler_params=pltpu.CompilerParams(
            dimension_semantics=("arbitrary",),
            vmem_limit_bytes=60 * 1024 * 1024,
        ),
    )(te, act, xs, w13, w13, w2)


def _comb_body(g0_ref, g1_ref, w_ref, o_ref):
    g0 = g0_ref[...].astype(jnp.float32)
    g1 = g1_ref[...].astype(jnp.float32)
    o_ref[...] = g0 * w_ref[:, 0:1] + g1 * w_ref[:, 1:2]


def _pair_combine(g, top_w, n, d):
    tb = 512
    nb = n // tb
    return pl.pallas_call(
        _comb_body,
        grid=(nb,),
        in_specs=[
            pl.BlockSpec((tb, d), lambda i: (i, 0)),
            pl.BlockSpec((tb, d), lambda i, _nb=nb: (i + _nb, 0)),
            pl.BlockSpec((tb, 2), lambda i: (i, 0)),
        ],
        out_specs=pl.BlockSpec((tb, d), lambda i: (i, 0)),
        out_shape=jax.ShapeDtypeStruct((n, d), jnp.float32),
    )(g, g, top_w)


def kernel(x, router_w, w13, w2):
    b, t, d = x.shape
    x_flat = x.reshape(-1, d)
    n = x_flat.shape[0]
    e = router_w.shape[0]
    n2 = 2 * n
    np_ = n2 + e * _TM
    nt = np_ // _TM

    top_w, p0c, p1c, ta = _router_meta(x_flat, router_w, nt)
    p0 = p0c.reshape(n)
    p1 = p1c.reshape(n)
    te = ta[:, 0]
    act = ta[:, 1]

    def _to_words(a):
        r = a.shape[0]
        return lax.bitcast_convert_type(a.reshape(r, d // 2, 2), jnp.int32)

    def _from_words(a):
        r = a.shape[0]
        return lax.bitcast_convert_type(a, jnp.bfloat16).reshape(r, d)

    x_bf = _to_words(x_flat.astype(jnp.bfloat16))
    xs = _from_words(_make_sc_dispatch(n, d, np_)(x_bf, p0, p1))
    slot_out = _ffn(xs, w13.astype(jnp.bfloat16), w2.astype(jnp.bfloat16),
                    te, act)
    pair_idx = jnp.concatenate([p0, p1])
    g = _from_words(
        _make_sc_gather(n2, d, np_)(_to_words(slot_out), pair_idx))
    out = _pair_combine(g, top_w, n, d)
    return out.reshape(b, t, d)


# speedup vs baseline: 2.7990x; 2.7990x over previous
import functools

import jax
import jax.numpy as jnp
from jax import lax
from jax.experimental import pallas as pl
from jax.experimental.pallas import tpu as pltpu
from jax.experimental.pallas import tpu_sc as plsc

_SC_CORES = 2
_SC_SUBCORES = 16
_NW = _SC_CORES * _SC_SUBCORES

_TM = 256


def _router_meta_body(x_ref, rw_ref, topw_ref, p0_ref, p1_ref, ta_ref,
                      idx_scr, rank_scr, carry_scr):
    ph = pl.program_id(0)
    i = pl.program_id(1)
    nb = pl.num_programs(1)
    tb = x_ref.shape[0]
    e = rw_ref.shape[0]

    @pl.when((ph == 0) & (i == 0))
    def _():
        carry_scr[...] = jnp.zeros_like(carry_scr)

    @pl.when(ph == 0)
    def _():
        xb = x_ref[...]
        logits = lax.dot_general(
            xb, rw_ref[...], (((1,), (1,)), ((), ())),
            preferred_element_type=jnp.float32)
        m = jnp.max(logits, axis=-1, keepdims=True)
        p = jnp.exp(logits - m)
        p = p / jnp.sum(p, axis=-1, keepdims=True)
        ids = lax.broadcasted_iota(jnp.int32, p.shape, 1)
        m1 = jnp.max(p, axis=-1, keepdims=True)
        i1 = jnp.min(jnp.where(p == m1, ids, e), axis=-1, keepdims=True)
        pm = jnp.where(ids == i1, -1.0, p)
        m2 = jnp.max(pm, axis=-1, keepdims=True)
        i2 = jnp.min(jnp.where(pm == m2, ids, e), axis=-1, keepdims=True)
        s = m1 + m2
        topw_ref[...] = jnp.concatenate([m1 / s, m2 / s], axis=1)
        oh = ((ids == i1) | (ids == i2)).astype(jnp.float32)
        r = lax.broadcasted_iota(jnp.int32, (tb, tb), 0)
        c = lax.broadcasted_iota(jnp.int32, (tb, tb), 1)
        tril = (r > c).astype(jnp.float32)
        pref = lax.dot_general(
            tril, oh, (((1,), (0,)), ((), ())),
            preferred_element_type=jnp.float32)
        prefc = pref + carry_scr[...]
        rank0 = jnp.sum(jnp.where(ids == i1, prefc, 0.0), axis=1,
                        keepdims=True)
        rank1 = jnp.sum(jnp.where(ids == i2, prefc, 0.0), axis=1,
                        keepdims=True)
        idx_scr[pl.ds(i * tb, tb), :] = jnp.concatenate([i1, i2], axis=1)
        rank_scr[pl.ds(i * tb, tb), :] = jnp.concatenate([rank0, rank1],
                                                         axis=1)
        carry_scr[...] += jnp.sum(oh, axis=0, keepdims=True)

    @pl.when(ph == 1)
    def _():
        counts = carry_scr[...]
        padded = jnp.ceil(counts / _TM) * _TM
        le = (lax.broadcasted_iota(jnp.int32, (e, e), 0)
              <= lax.broadcasted_iota(jnp.int32, (e, e), 1)
              ).astype(jnp.float32)
        pend = lax.dot_general(
            padded, le, (((1,), (0,)), ((), ())),
            preferred_element_type=jnp.float32)
        pstart = pend - padded
        tpi = idx_scr[pl.ds(i * tb, tb), :]
        rk = rank_scr[pl.ds(i * tb, tb), :]
        ids = lax.broadcasted_iota(jnp.int32, (tb, e), 1)
        ps0 = jnp.sum(jnp.where(ids == tpi[:, 0:1], pstart, 0.0), axis=1,
                      keepdims=True)
        ps1 = jnp.sum(jnp.where(ids == tpi[:, 1:2], pstart, 0.0), axis=1,
                      keepdims=True)
        p0_ref[...] = (ps0 + rk[:, 0:1]).astype(jnp.int32)
        p1_ref[...] = (ps1 + rk[:, 1:2]).astype(jnp.int32)

        @pl.when(i == nb - 1)
        def _():
            ntt = ta_ref.shape[0]
            basev = (lax.broadcasted_iota(jnp.int32, (ntt, 1), 0)
                     * _TM).astype(jnp.float32)
            cmp = pend <= basev
            te = jnp.minimum(jnp.sum(cmp.astype(jnp.int32), axis=1,
                                     keepdims=True), e - 1)
            act = (basev < jnp.max(pend)).astype(jnp.int32)
            ta_ref[...] = jnp.concatenate([te, act], axis=1)


def _router_meta(x_flat, router_w, nt):
    n, d = x_flat.shape
    e = router_w.shape[0]
    tb = 512
    nb = n // tb
    return pl.pallas_call(
        _router_meta_body,
        grid=(2, nb),
        in_specs=[
            pl.BlockSpec((tb, d), lambda p, i: ((1 - p) * i, 0)),
            pl.BlockSpec((e, d), lambda p, i: (0, 0)),
        ],
        out_specs=[
            pl.BlockSpec((tb, 2), lambda p, i, _nb=nb: ((1 - p) * i + p * (_nb - 1), 0)),
            pl.BlockSpec((tb, 1), lambda p, i: (p * i, 0)),
            pl.BlockSpec((tb, 1), lambda p, i: (p * i, 0)),
            pl.BlockSpec((nt, 2), lambda p, i: (0, 0)),
        ],
        out_shape=[
            jax.ShapeDtypeStruct((n, 2), jnp.float32),
            jax.ShapeDtypeStruct((n, 1), jnp.int32),
            jax.ShapeDtypeStruct((n, 1), jnp.int32),
            jax.ShapeDtypeStruct((nt, 2), jnp.int32),
        ],
        scratch_shapes=[
            pltpu.VMEM((n, 2), jnp.int32),
            pltpu.VMEM((n, 2), jnp.float32),
            pltpu.VMEM((1, e), jnp.float32),
        ],
        compiler_params=pltpu.CompilerParams(
            dimension_semantics=("arbitrary", "arbitrary"),
        ),
    )(x_flat, router_w)


def _make_sc_gather(rows_out, d, table_rows):
    per_w = rows_out // _NW
    ch = per_w
    while ch > 48 or per_w % ch:
        ch -= 8
    n_ch = per_w // ch
    mesh = plsc.VectorSubcoreMesh(
        core_axis_name="c", subcore_axis_name="s",
        num_cores=_SC_CORES, num_subcores=_SC_SUBCORES,
    )

    @functools.partial(
        pl.kernel,
        out_type=jax.ShapeDtypeStruct((rows_out, d), jnp.float32),
        mesh=mesh,
        scratch_types=[
            pltpu.VMEM((2, ch), jnp.int32),
            pltpu.VMEM((2, ch, d), jnp.float32),
            pltpu.SemaphoreType.DMA((2,)),
        ],
    )
    def gather(table_hbm, idx_hbm, out_hbm, idx_v, rows_v, sems):
        wid = lax.axis_index("s") * _SC_CORES + lax.axis_index("c")
        base = wid * per_w

        def start(c):
            b = c % 2
            pltpu.sync_copy(idx_hbm.at[pl.ds(base + c * ch, ch)], idx_v.at[b])
            return pltpu.async_copy(table_hbm.at[idx_v.at[b]], rows_v.at[b],
                                    sems.at[b])

        handles = [start(0)]
        for c in range(n_ch):
            if c + 1 < n_ch:
                handles.append(start(c + 1))
            handles[c].wait()
            pltpu.sync_copy(rows_v.at[c % 2],
                            out_hbm.at[pl.ds(base + c * ch, ch)])

    return gather


def _make_sc_dispatch(n, d, np_):
    per_w = n // _NW
    ch = per_w
    while ch > 48 or per_w % ch:
        ch -= 8
    n_ch = per_w // ch
    mesh = plsc.VectorSubcoreMesh(
        core_axis_name="c", subcore_axis_name="s",
        num_cores=_SC_CORES, num_subcores=_SC_SUBCORES,
    )

    @functools.partial(
        pl.kernel,
        out_type=jax.ShapeDtypeStruct((np_, d), jnp.float32),
        mesh=mesh,
        scratch_types=[
            pltpu.VMEM((2, ch), jnp.int32),
            pltpu.VMEM((2, ch), jnp.int32),
            pltpu.VMEM((2, ch, d), jnp.float32),
            pltpu.SemaphoreType.DMA((2,)),
        ],
    )
    def dispatch(x_hbm, p0_hbm, p1_hbm, out_hbm, i0_v, i1_v, rows_v, sems):
        wid = lax.axis_index("s") * _SC_CORES + lax.axis_index("c")
        base = wid * per_w

        def start(c):
            b = c % 2
            sl = pl.ds(base + c * ch, ch)
            pltpu.sync_copy(x_hbm.at[sl], rows_v.at[b])
            pltpu.sync_copy(p0_hbm.at[sl], i0_v.at[b])
            pltpu.sync_copy(p1_hbm.at[sl], i1_v.at[b])
            h0 = pltpu.async_copy(rows_v.at[b], out_hbm.at[i0_v.at[b]],
                                  sems.at[b])
            h1 = pltpu.async_copy(rows_v.at[b], out_hbm.at[i1_v.at[b]],
                                  sems.at[b])
            return h0, h1

        handles = [start(0)]
        for c in range(n_ch):
            if c + 1 < n_ch:
                handles.append(start(c + 1))
            h0, h1 = handles[c]
            h0.wait()
            h1.wait()

    return dispatch


def _ffn_half_body(te_ref, act_ref, xs_ref, w1_ref, w3_ref, w2_ref, *rest):
    if len(rest) == 5:
        prev_ref, out_ref, w1s, w3s, w2s = rest
    else:
        prev_ref = None
        out_ref, w1s, w3s, w2s = rest
    i = pl.program_id(0)
    changed = (i == 0) | (te_ref[i] != te_ref[jnp.maximum(i - 1, 0)])

    @pl.when(changed)
    def _():
        w1s[...] = w1_ref[0].astype(jnp.bfloat16)
        w3s[...] = w3_ref[0].astype(jnp.bfloat16)
        w2s[...] = w2_ref[0].astype(jnp.bfloat16)

    @pl.when(act_ref[i] != 0)
    def _():
        xb = xs_ref[...].astype(jnp.bfloat16)
        h1 = lax.dot_general(
            xb, w1s[...], (((1,), (1,)), ((), ())),
            preferred_element_type=jnp.float32)
        h3 = lax.dot_general(
            xb, w3s[...], (((1,), (1,)), ((), ())),
            preferred_element_type=jnp.float32)
        g = ((h1 * jax.nn.sigmoid(h1)) * h3).astype(jnp.bfloat16)
        acc = lax.dot_general(
            g, w2s[...], (((1,), (0,)), ((), ())),
            preferred_element_type=jnp.float32)
        if prev_ref is not None:
            acc = acc + prev_ref[...]
        out_ref[...] = acc


def _ffn_half(xs, w13, w2t, te, act, half, prev=None):
    np_, d = xs.shape
    e, f2, _ = w13.shape
    fh = f2 // 4
    nt = np_ // _TM
    in_specs = [
        pl.BlockSpec((_TM, d), lambda i, te, act: (i, 0)),
        pl.BlockSpec((1, fh, d),
                     lambda i, te, act, _h=half: (te[i], _h, 0)),
        pl.BlockSpec((1, fh, d),
                     lambda i, te, act, _h=half: (te[i], 2 + _h, 0)),
        pl.BlockSpec((1, fh, d),
                     lambda i, te, act, _h=half: (te[i], _h, 0)),
    ]
    args = [te, act, xs, w13, w13, w2t]
    if prev is not None:
        in_specs.append(pl.BlockSpec((_TM, d), lambda i, te, act: (i, 0)))
        args.append(prev)
    grid_spec = pltpu.PrefetchScalarGridSpec(
        num_scalar_prefetch=2,
        grid=(nt,),
        in_specs=in_specs,
        out_specs=pl.BlockSpec((_TM, d), lambda i, te, act: (i, 0)),
        scratch_shapes=[
            pltpu.VMEM((fh, d), jnp.bfloat16),
            pltpu.VMEM((fh, d), jnp.bfloat16),
            pltpu.VMEM((fh, d), jnp.bfloat16),
        ],
    )
    return pl.pallas_call(
        _ffn_half_body,
        grid_spec=grid_spec,
        out_shape=jax.ShapeDtypeStruct((np_, d), jnp.float32),
        compiler_params=pltpu.CompilerParams(
            dimension_semantics=("arbitrary",),
            vmem_limit_bytes=60 * 1024 * 1024,
        ),
    )(*args)


def _comb_body(g0_ref, g1_ref, w_ref, o_ref):
    o_ref[...] = g0_ref[...] * w_ref[:, 0:1] + g1_ref[...] * w_ref[:, 1:2]


def _pair_combine(g, top_w, n, d):
    tb = 512
    nb = n // tb
    return pl.pallas_call(
        _comb_body,
        grid=(nb,),
        in_specs=[
            pl.BlockSpec((tb, d), lambda i: (i, 0)),
            pl.BlockSpec((tb, d), lambda i, _nb=nb: (i + _nb, 0)),
            pl.BlockSpec((tb, 2), lambda i: (i, 0)),
        ],
        out_specs=pl.BlockSpec((tb, d), lambda i: (i, 0)),
        out_shape=jax.ShapeDtypeStruct((n, d), jnp.float32),
    )(g, g, top_w)


def kernel(x, router_w, w13, w2):
    b, t, d = x.shape
    x_flat = x.reshape(-1, d)
    n = x_flat.shape[0]
    e = router_w.shape[0]
    n2 = 2 * n
    np_ = n2 + e * _TM
    nt = np_ // _TM

    top_w, p0c, p1c, ta = _router_meta(x_flat, router_w, nt)
    p0 = p0c.reshape(n)
    p1 = p1c.reshape(n)
    te = ta[:, 0]
    act = ta[:, 1]

    xs = _make_sc_dispatch(n, d, np_)(x_flat, p0, p1)
    w2t = jnp.swapaxes(w2, 1, 2)
    half0 = _ffn_half(xs, w13, w2t, te, act, 0)
    slot_out = _ffn_half(xs, w13, w2t, te, act, 1, prev=half0)
    pair_idx = jnp.concatenate([p0, p1])
    g = _make_sc_gather(n2, d, np_)(slot_out, pair_idx)
    out = _pair_combine(g, top_w, n, d)
    return out.reshape(b, t, d)
